# baseline (speedup 1.0000x reference)
import jax
import jax.numpy as jnp
from jax import lax
from jax.experimental import pallas as pl
from jax.experimental.pallas import tpu as pltpu
from jax.experimental.pallas import tpu_sc as plsc

N = 10000
D_IN = 128
HID = 16
NCLS = 128

NC = 2
NS = 16
NW = NC * NS
CH = 128
ECH = 2500
WCH = ECH // NW
TAILW = ECH - WCH * NW
NPAD = 10112
STRIPE = NPAD // NS
BLK = NPAD // 8
NBLK = N // 8
SEGN = 6
SEGC = WCH // SEGN

_mesh = plsc.VectorSubcoreMesh(core_axis_name="c", subcore_axis_name="s")
_sc_params = pltpu.CompilerParams(use_tc_tiling_on_sc=False)


def _zero_stripe(zbuf, acc, s):
    def zrow(i, _):
        zbuf[i] = jnp.zeros((HID,), jnp.float32)
        return 0
    lax.fori_loop(0, STRIPE, zrow, 0)
    pltpu.sync_copy(zbuf, acc.at[pl.ds(s * STRIPE, STRIPE)])


def _writeback(acc, out, c, s):
    pltpu.sync_copy(acc.at[pl.ds(s * STRIPE, STRIPE)],
                    out.at[c, pl.ds(s * STRIPE, STRIPE)])


WCHE = WCH * CH
EW0 = NW * WCHE
SEGE = SEGC * CH


def _stage_one(m, v, wid):
    pltpu.sync_copy(m.at[pl.ds(wid * WCHE, WCHE)], v.at[pl.ds(0, WCHE)])

    @pl.when(wid < TAILW)
    def _tail():
        pltpu.sync_copy(m.at[pl.ds(EW0 + wid * CH, CH)],
                        v.at[pl.ds(WCHE, CH)])


def _stage_indices(srcm, dstm, src_v, dst_v, wid):
    _stage_one(srcm, src_v, wid)
    _stage_one(dstm, dst_v, wid)


def _segsum_body(table, em, out, src_v, dst_v, rows, zbuf, acc,
                 g0, g1, s0, s1):
    gsems = (g0, g1)
    ssems = (s0, s1)
    srcm = em.at[0]
    dstm = em.at[1]
    c = lax.axis_index("c")
    s = lax.axis_index("s")
    wid = s * NC + c
    _stage_indices(srcm, dstm, src_v, dst_v, wid)
    _zero_stripe(zbuf, acc, s)
    plsc.subcore_barrier()

    def gidx(g):
        return src_v.at[pl.ds(g * SEGE, SEGE)]

    def didx(g):
        return dst_v.at[pl.ds(g * SEGE, SEGE)]

    def gather(g, b):
        return pltpu.async_copy(table.at[gidx(g)], rows.at[b], gsems[b])

    def scatter(g, b):
        return pltpu.async_copy(rows.at[b], acc.at[didx(g)], ssems[b],
                                add=True)

    gather(0, 0)
    for g in range(SEGN):
        b = g % 2
        pltpu.make_async_copy(table.at[gidx(g)], rows.at[b],
                              gsems[b]).wait()
        if g + 1 < SEGN:
            if g >= 1:
                pltpu.make_async_copy(rows.at[1 - b], acc.at[didx(g - 1)],
                                      ssems[1 - b]).wait()
            gather(g + 1, 1 - b)
        scatter(g, b)
    pltpu.make_async_copy(rows.at[0], acc.at[didx(SEGN - 2)],
                          ssems[0]).wait()
    pltpu.make_async_copy(rows.at[1], acc.at[didx(SEGN - 1)],
                          ssems[1]).wait()

    @pl.when(wid < TAILW)
    def _tail():
        tb = rows.at[0].at[pl.ds(0, CH)]
        pltpu.async_copy(table.at[src_v.at[pl.ds(WCHE, CH)]], tb,
                         gsems[0]).wait()
        pltpu.sync_copy(tb, acc.at[dst_v.at[pl.ds(WCHE, CH)]], add=True)

    plsc.subcore_barrier()
    _writeback(acc, out, c, s)


_segsum = pl.kernel(
    _segsum_body,
    out_type=jax.ShapeDtypeStruct((NC, NPAD, HID), jnp.float32),
    mesh=_mesh,
    scratch_types=[
        pltpu.VMEM((WCHE + CH,), jnp.int32),
        pltpu.VMEM((WCHE + CH,), jnp.int32),
        pltpu.VMEM((2, SEGE, HID), jnp.float32),
        pltpu.VMEM((STRIPE, HID), jnp.float32),
        pltpu.VMEM_SHARED((NPAD, HID), jnp.float32),
    ] + [pltpu.SemaphoreType.DMA] * 4,
    compiler_params=_sc_params,
)


def _deg_body(em, out, dst_v, ones_v, zbuf, acc, dsem):
    c = lax.axis_index("c")
    s = lax.axis_index("s")
    wid = s * NC + c
    _stage_one(em.at[1], dst_v, wid)

    def orow(i, _):
        ones_v[i] = jnp.ones((HID,), jnp.float32)
        return 0
    lax.fori_loop(0, SEGE, orow, 0)
    _zero_stripe(zbuf, acc, s)
    plsc.subcore_barrier()

    for g in range(SEGN):
        didx = dst_v.at[pl.ds(g * SEGE, SEGE)]
        pltpu.async_copy(ones_v, acc.at[didx], dsem, add=True)

    tidx = dst_v.at[pl.ds(WCHE, CH)]
    tones = ones_v.at[pl.ds(0, CH)]

    @pl.when(wid < TAILW)
    def _tail():
        pltpu.async_copy(tones, acc.at[tidx], dsem, add=True)

    for g in range(SEGN):
        didx = dst_v.at[pl.ds(g * SEGE, SEGE)]
        pltpu.make_async_copy(ones_v, acc.at[didx], dsem).wait()

    @pl.when(wid < TAILW)
    def _tail_wait():
        pltpu.make_async_copy(tones, acc.at[tidx], dsem).wait()

    plsc.subcore_barrier()
    _writeback(acc, out, c, s)


_deg = pl.kernel(
    _deg_body,
    out_type=jax.ShapeDtypeStruct((NC, NPAD, HID), jnp.float32),
    mesh=_mesh,
    scratch_types=[
        pltpu.VMEM((WCHE + CH,), jnp.int32),
        pltpu.VMEM((SEGE, HID), jnp.float32),
        pltpu.VMEM((STRIPE, HID), jnp.float32),
        pltpu.VMEM_SHARED((NPAD, HID), jnp.float32),
        pltpu.SemaphoreType.DMA,
    ],
    compiler_params=_sc_params,
)


def _tcz_body(xv_ref, w1b_ref, z_ref):
    z_ref[...] = jnp.dot(xv_ref[...], w1b_ref[...],
                         preferred_element_type=jnp.float32)


_tcz = pl.pallas_call(
    _tcz_body,
    out_shape=jax.ShapeDtypeStruct((NBLK, 128), jnp.float32),
)


def _tc1_body(deg_ref, z_ref, dinv_ref, as1_ref):
    deg = deg_ref[0] + deg_ref[1] + 1.0
    dinv = lax.rsqrt(deg)
    dinv_ref[...] = dinv
    as1_ref[0:NBLK, :] = z_ref[...] * dinv[0:NBLK, :]


_tc1 = pl.pallas_call(
    _tc1_body,
    out_shape=[
        jax.ShapeDtypeStruct((BLK, 128), jnp.float32),
        jax.ShapeDtypeStruct((BLK, 128), jnp.float32),
    ],
)


def _tc2_body(s_ref, t_ref, dinv_ref, wb_ref, b_ref, as_ref):
    dinv = dinv_ref[...]
    h = (s_ref[0] + s_ref[1] + t_ref[...]) * dinv + b_ref[...]
    z = jnp.dot(h, wb_ref[...], preferred_element_type=jnp.float32)
    as_ref[...] = z * dinv


_tc2 = pl.pallas_call(
    _tc2_body,
    out_shape=jax.ShapeDtypeStruct((BLK, 128), jnp.float32),
)


def _tc3_body(s_ref, t_ref, dinv_ref, b_ref, as_ref):
    dinv = dinv_ref[...]
    as_ref[...] = ((s_ref[0] + s_ref[1] + t_ref[...]) * dinv
                   + b_ref[...]) * dinv


_tc3 = pl.pallas_call(
    _tc3_body,
    out_shape=jax.ShapeDtypeStruct((BLK, 128), jnp.float32),
)


def _tc4a_body(s_ref, t_ref, dinv_ref, g_ref):
    g_ref[...] = ((s_ref[0, 0:NBLK, :] + s_ref[1, 0:NBLK, :]
                   + t_ref[0:NBLK, :]) * dinv_ref[0:NBLK, :])


_tc4a = pl.pallas_call(
    _tc4a_body,
    out_shape=jax.ShapeDtypeStruct((NBLK, 128), jnp.float32),
)


def _tc4b_body(g_ref, w3_ref, b3_ref, lw_ref, lb_ref, out_ref):
    h3 = jnp.dot(g_ref[...], w3_ref[...],
                 preferred_element_type=jnp.float32)
    h3 = h3 + b3_ref[...]
    logits = jnp.dot(h3, lw_ref[...], preferred_element_type=jnp.float32)
    logits = logits + lb_ref[...]
    m = jnp.max(logits, axis=1, keepdims=True)
    e = jnp.exp(logits - m)
    lse = jnp.log(jnp.sum(e, axis=1, keepdims=True)) + m
    out_ref[...] = logits - lse


_tc4b = pl.pallas_call(
    _tc4b_body,
    out_shape=jax.ShapeDtypeStruct((N, NCLS), jnp.float32),
)


def kernel(x, edge_index, W1, b1, W2, b2, W3, b3, lin_W, lin_b):
    em = edge_index.astype(jnp.int32)
    xv = x.reshape(NBLK, 8 * D_IN)

    ey8 = jnp.eye(8, dtype=jnp.float32)
    w1b = jnp.kron(ey8, W1)
    w2b = jnp.kron(ey8, W2)
    b1t = jnp.tile(b1, 8).reshape(1, 128)
    b2t = jnp.tile(b2, 8).reshape(1, 128)

    def blk(a):
        return a.reshape(NC, BLK, 128)

    z8 = _tcz(xv, w1b)
    deg2 = blk(_deg(em))
    dinv, as1 = _tc1(deg2, z8)
    s1 = blk(_segsum(as1.reshape(NPAD, HID), em))
    as2 = _tc2(s1, as1, dinv, w2b, b1t)
    s2 = blk(_segsum(as2.reshape(NPAD, HID), em))
    as3 = _tc3(s2, as2, dinv, b2t)
    s3 = blk(_segsum(as3.reshape(NPAD, HID), em))
    g8 = _tc4a(s3, as3, dinv)
    out = _tc4b(g8.reshape(N, HID), W3, b3.reshape(1, NCLS),
                lin_W, lin_b.reshape(1, NCLS))
    return out


# speedup vs baseline: 1.1202x; 1.1202x over previous
import jax
import jax.numpy as jnp
from jax import lax
from jax.experimental import pallas as pl
from jax.experimental.pallas import tpu as pltpu
from jax.experimental.pallas import tpu_sc as plsc

N = 10000
D_IN = 128
HID = 16
NCLS = 128

NC = 2
NS = 16
NW = NC * NS
CH = 128
ECH = 2500
WCH = ECH // NW
TAILW = ECH - WCH * NW
NPAD = 10112
STRIPE = NPAD // NS
BLK = NPAD // 8
NBLK = N // 8
NBUF = 26
GRP = WCH // NBUF

_mesh = plsc.VectorSubcoreMesh(core_axis_name="c", subcore_axis_name="s")
_sc_params = pltpu.CompilerParams(use_tc_tiling_on_sc=False)


def _zero_stripe(zbuf, acc, s):
    def zrow(i, _):
        zbuf[i] = jnp.zeros((HID,), jnp.float32)
        return 0
    lax.fori_loop(0, STRIPE, zrow, 0)
    pltpu.sync_copy(zbuf, acc.at[pl.ds(s * STRIPE, STRIPE)])


def _writeback(acc, out, c, s):
    pltpu.sync_copy(acc.at[pl.ds(s * STRIPE, STRIPE)],
                    out.at[c, pl.ds(s * STRIPE, STRIPE)])


def _stage_one(m, v, wid):
    pltpu.sync_copy(m.at[pl.ds(wid * WCH, WCH)], v.at[pl.ds(0, WCH)])

    @pl.when(wid < TAILW)
    def _tail():
        pltpu.sync_copy(m.at[pl.ds(NW * WCH + wid, 1)], v.at[pl.ds(WCH, 1)])


def _segsum_body(table, em, out, src_v, dst_v, rows, zbuf, acc, *sems):
    gsems = sems[:NBUF]
    ssem = sems[NBUF]
    c = lax.axis_index("c")
    s = lax.axis_index("s")
    wid = s * NC + c
    _stage_one(em.at[0], src_v, wid)
    _stage_one(em.at[1], dst_v, wid)
    _zero_stripe(zbuf, acc, s)
    plsc.subcore_barrier()

    for b in range(NBUF):
        pltpu.async_copy(table.at[src_v.at[b]], rows.at[b], gsems[b])

    dummy = table.at[pl.ds(0, CH)]

    def grp_body(g, _):
        base = g * NBUF
        for b in range(NBUF):
            pltpu.make_async_copy(dummy, rows.at[b], gsems[b]).wait()
            pltpu.async_copy(rows.at[b], acc.at[dst_v.at[base + b]],
                             ssem, add=True)
        for b in range(NBUF):
            pltpu.make_async_copy(dummy, rows.at[b], ssem).wait()
        for b in range(NBUF):
            nxt = base + NBUF + b

            @pl.when(nxt < WCH)
            def _fire():
                pltpu.async_copy(table.at[src_v.at[nxt]], rows.at[b],
                                 gsems[b])
        return 0
    lax.fori_loop(0, GRP, grp_body, 0)

    @pl.when(wid < TAILW)
    def _tail():
        pltpu.async_copy(table.at[src_v.at[WCH]], rows.at[0],
                         gsems[0]).wait()
        pltpu.sync_copy(rows.at[0], acc.at[dst_v.at[WCH]], add=True)

    plsc.subcore_barrier()
    _writeback(acc, out, c, s)


_segsum = pl.kernel(
    _segsum_body,
    out_type=jax.ShapeDtypeStruct((NC, NPAD, HID), jnp.float32),
    mesh=_mesh,
    scratch_types=[
        pltpu.VMEM((WCH + 1, CH), jnp.int32),
        pltpu.VMEM((WCH + 1, CH), jnp.int32),
        pltpu.VMEM((NBUF, CH, HID), jnp.float32),
        pltpu.VMEM((STRIPE, HID), jnp.float32),
        pltpu.VMEM_SHARED((NPAD, HID), jnp.float32),
    ] + [pltpu.SemaphoreType.DMA] * (NBUF + 1),
    compiler_params=_sc_params,
)


def _deg_body(em, out, dst_v, ones_v, zbuf, acc, dsem):
    c = lax.axis_index("c")
    s = lax.axis_index("s")
    wid = s * NC + c
    _stage_one(em.at[1], dst_v, wid)

    def orow(i, _):
        ones_v[i] = jnp.ones((HID,), jnp.float32)
        return 0
    lax.fori_loop(0, CH, orow, 0)
    _zero_stripe(zbuf, acc, s)
    plsc.subcore_barrier()

    ncw = WCH + jnp.where(wid < TAILW, 1, 0)

    def chunk(j, _):
        pltpu.async_copy(ones_v, acc.at[dst_v.at[j]], dsem, add=True)
        return 0
    lax.fori_loop(0, ncw, chunk, 0)

    dummy = out.at[0, pl.ds(0, CH)]

    def drain(j, _):
        pltpu.make_async_copy(dummy, ones_v, dsem).wait()
        return 0
    lax.fori_loop(0, ncw, drain, 0)

    plsc.subcore_barrier()
    _writeback(acc, out, c, s)


_deg = pl.kernel(
    _deg_body,
    out_type=jax.ShapeDtypeStruct((NC, NPAD, HID), jnp.float32),
    mesh=_mesh,
    scratch_types=[
        pltpu.VMEM((WCH + 1, CH), jnp.int32),
        pltpu.VMEM((CH, HID), jnp.float32),
        pltpu.VMEM((STRIPE, HID), jnp.float32),
        pltpu.VMEM_SHARED((NPAD, HID), jnp.float32),
        pltpu.SemaphoreType.DMA,
    ],
    compiler_params=_sc_params,
)


def _tcz_body(xv_ref, w1b_ref, z_ref):
    z_ref[...] = jnp.dot(xv_ref[...], w1b_ref[...],
                         preferred_element_type=jnp.float32)


_tcz = pl.pallas_call(
    _tcz_body,
    out_shape=jax.ShapeDtypeStruct((NBLK, 128), jnp.float32),
)


def _tc1_body(deg_ref, z_ref, dinv_ref, as1_ref):
    deg = deg_ref[0] + deg_ref[1] + 1.0
    dinv = lax.rsqrt(deg)
    dinv_ref[...] = dinv
    as1_ref[0:NBLK, :] = z_ref[...] * dinv[0:NBLK, :]


_tc1 = pl.pallas_call(
    _tc1_body,
    out_shape=[
        jax.ShapeDtypeStruct((BLK, 128), jnp.float32),
        jax.ShapeDtypeStruct((BLK, 128), jnp.float32),
    ],
)


def _tc2_body(s_ref, t_ref, dinv_ref, wb_ref, b_ref, as_ref):
    dinv = dinv_ref[...]
    h = (s_ref[0] + s_ref[1] + t_ref[...]) * dinv + b_ref[...]
    z = jnp.dot(h, wb_ref[...], preferred_element_type=jnp.float32)
    as_ref[...] = z * dinv


_tc2 = pl.pallas_call(
    _tc2_body,
    out_shape=jax.ShapeDtypeStruct((BLK, 128), jnp.float32),
)


def _tc3_body(s_ref, t_ref, dinv_ref, b_ref, as_ref):
    dinv = dinv_ref[...]
    as_ref[...] = ((s_ref[0] + s_ref[1] + t_ref[...]) * dinv
                   + b_ref[...]) * dinv


_tc3 = pl.pallas_call(
    _tc3_body,
    out_shape=jax.ShapeDtypeStruct((BLK, 128), jnp.float32),
)


def _tc4_body(s_ref, t_ref, dinv_ref, w3_ref, b3_ref, lw_ref, lb_ref,
              out_ref):
    g8 = ((s_ref[0, 0:NBLK, :] + s_ref[1, 0:NBLK, :] + t_ref[0:NBLK, :])
          * dinv_ref[0:NBLK, :])
    lw = lw_ref[...]
    for i in range(8):
        gi = g8[:, HID * i:HID * (i + 1)]
        h3 = jnp.dot(gi, w3_ref[...], preferred_element_type=jnp.float32)
        h3 = h3 + b3_ref[...]
        logits = jnp.dot(h3, lw, preferred_element_type=jnp.float32)
        logits = logits + lb_ref[...]
        m = jnp.max(logits, axis=1, keepdims=True)
        e = jnp.exp(logits - m)
        lse = jnp.log(jnp.sum(e, axis=1, keepdims=True)) + m
        out_ref[:, i, :] = logits - lse


_tc4 = pl.pallas_call(
    _tc4_body,
    out_shape=jax.ShapeDtypeStruct((NBLK, 8, NCLS), jnp.float32),
)


def kernel(x, edge_index, W1, b1, W2, b2, W3, b3, lin_W, lin_b):
    em = edge_index.astype(jnp.int32).reshape(2, ECH, CH)
    xv = x.reshape(NBLK, 8 * D_IN)

    ey8 = jnp.eye(8, dtype=jnp.float32)
    w1b = jnp.kron(ey8, W1)
    w2b = jnp.kron(ey8, W2)
    b1t = jnp.tile(b1, 8).reshape(1, 128)
    b2t = jnp.tile(b2, 8).reshape(1, 128)

    def blk(a):
        return a.reshape(NC, BLK, 128)

    z8 = _tcz(xv, w1b)
    deg2 = blk(_deg(em))
    dinv, as1 = _tc1(deg2, z8)
    s1 = blk(_segsum(as1.reshape(NPAD, HID), em))
    as2 = _tc2(s1, as1, dinv, w2b, b1t)
    s2 = blk(_segsum(as2.reshape(NPAD, HID), em))
    as3 = _tc3(s2, as2, dinv, b2t)
    s3 = blk(_segsum(as3.reshape(NPAD, HID), em))
    out = _tc4(s3, as3, dinv, W3, b3.reshape(1, NCLS),
               lin_W, lin_b.reshape(1, NCLS))
    return out.reshape(N, NCLS)
